# baseline (speedup 1.0000x reference)
import functools

import jax
import jax.numpy as jnp
from jax import lax
from jax.experimental import pallas as pl
from jax.experimental.pallas import tpu as pltpu
from jax.experimental.pallas import tpu_sc as plsc

BATCH = 16384
FEAT = 64
LANES = 16
NUM_CORES = 2
NUM_SUBCORES = 16
NUM_WORKERS = NUM_CORES * NUM_SUBCORES
BPW = BATCH // NUM_WORKERS
CHUNK = 128
NCHUNK = BPW // CHUNK
VECS_PER_ROW = FEAT // LANES


def _body(feat_hbm, lab_hbm, cent_hbm, out_hbm, idx_v, cent_v, feat_v,
          acc_v, gsem, fsem):
    wid = lax.axis_index("s") * NUM_CORES + lax.axis_index("c")
    pltpu.sync_copy(lab_hbm.at[wid], idx_v)
    fcopy = pltpu.async_copy(feat_hbm.at[pl.ds(wid * BPW, BPW)], feat_v, fsem)
    copies = []
    for j in range(NCHUNK):
        copies.append(pltpu.async_copy(
            cent_hbm.at[idx_v.at[j]],
            cent_v.at[pl.ds(j * CHUNK, CHUNK)], gsem))
    for c in copies:
        c.wait()
    fcopy.wait()

    zero = jnp.zeros((LANES,), jnp.float32)

    def row_step(i, accs):
        out = []
        for l in range(VECS_PER_ROW):
            d = feat_v[i, pl.ds(l * LANES, LANES)] - cent_v[i, pl.ds(l * LANES, LANES)]
            out.append(accs[l] + d * d)
        return tuple(out)

    accs = lax.fori_loop(0, BPW, row_step, (zero,) * VECS_PER_ROW)
    total = accs[0] + accs[1] + accs[2] + accs[3]
    acc_v[...] = total
    pltpu.sync_copy(acc_v, out_hbm.at[wid])


@functools.partial(jax.jit, static_argnames=())
def _center_loss(features, labels, centers):
    labels = labels.astype(jnp.int32).reshape(NUM_WORKERS, NCHUNK, CHUNK)
    kern = pl.kernel(
        _body,
        out_type=jax.ShapeDtypeStruct((NUM_WORKERS, LANES), jnp.float32),
        mesh=plsc.VectorSubcoreMesh(core_axis_name="c", subcore_axis_name="s"),
        scratch_types=[
            pltpu.VMEM((NCHUNK, CHUNK), jnp.int32),
            pltpu.VMEM((BPW, FEAT), jnp.float32),
            pltpu.VMEM((BPW, FEAT), jnp.float32),
            pltpu.VMEM((LANES,), jnp.float32),
            pltpu.SemaphoreType.DMA,
            pltpu.SemaphoreType.DMA,
        ],
        compiler_params=pltpu.CompilerParams(use_tc_tiling_on_sc=False),
    )
    partials = kern(features, labels, centers)
    return jnp.sum(partials) / BATCH


def kernel(features, labels, centers):
    return _center_loss(features, labels, centers)


# speedup vs baseline: 1.6773x; 1.6773x over previous
import functools

import jax
import jax.numpy as jnp
from jax import lax
from jax.experimental import pallas as pl
from jax.experimental.pallas import tpu as pltpu
from jax.experimental.pallas import tpu_sc as plsc

BATCH = 16384
FEAT = 64
LANES = 16
NUM_CORES = 2
NUM_SUBCORES = 16
NUM_WORKERS = NUM_CORES * NUM_SUBCORES
BPW = BATCH // NUM_WORKERS
CHB = 64
NCH = BPW // CHB
VECS_PER_ROW = FEAT // LANES


def _body(feat_hbm, lab_hbm, cent_hbm, out_hbm, idx_v, blk_v, feat_v,
          acc_v, gsems, fsem):
    wid = lax.axis_index("s") * NUM_CORES + lax.axis_index("c")
    base = wid * BPW
    pltpu.sync_copy(lab_hbm.at[pl.ds(base, BPW)], idx_v)
    fcopy = pltpu.async_copy(feat_hbm.at[pl.ds(base, BPW), :], feat_v, fsem)

    def fire(c, buf):
        def issue(v, _):
            labv = idx_v[pl.ds(c * CHB + v * LANES, LANES)]
            for k in range(LANES):
                pltpu.async_copy(cent_hbm.at[labv[k]],
                                 blk_v.at[buf, v * LANES + k],
                                 gsems.at[buf])
            return 0
        lax.fori_loop(0, CHB // LANES, issue, 0)

    def drain(buf):
        pltpu.make_async_copy(cent_hbm.at[pl.ds(0, CHB), :],
                              blk_v.at[buf], gsems.at[buf]).wait()

    def compute(c, buf, accs):
        def row(i, a):
            g = c * CHB + i
            out = []
            for l in range(VECS_PER_ROW):
                d = (feat_v[g, pl.ds(l * LANES, LANES)]
                     - blk_v[buf, i, pl.ds(l * LANES, LANES)])
                out.append(a[l] + d * d)
            return tuple(out)
        return lax.fori_loop(0, CHB, row, accs)

    zero = jnp.zeros((LANES,), jnp.float32)
    accs = (zero,) * VECS_PER_ROW
    fire(0, 0)
    fire(1, 1)
    fcopy.wait()
    for c in range(NCH):
        drain(c % 2)
        accs = compute(c, c % 2, accs)
        if c + 2 < NCH:
            fire(c + 2, c % 2)

    total = accs[0] + accs[1] + accs[2] + accs[3]
    acc_v[...] = total
    pltpu.sync_copy(acc_v, out_hbm.at[wid])


@functools.partial(jax.jit, static_argnames=())
def _center_loss(features, labels, centers):
    labels = labels.astype(jnp.int32)
    kern = pl.kernel(
        _body,
        out_type=jax.ShapeDtypeStruct((NUM_WORKERS, LANES), jnp.float32),
        mesh=plsc.VectorSubcoreMesh(core_axis_name="c", subcore_axis_name="s"),
        scratch_types=[
            pltpu.VMEM((BPW,), jnp.int32),
            pltpu.VMEM((2, CHB, FEAT), jnp.float32),
            pltpu.VMEM((BPW, FEAT), jnp.float32),
            pltpu.VMEM((LANES,), jnp.float32),
            pltpu.SemaphoreType.DMA((2,)),
            pltpu.SemaphoreType.DMA,
        ],
    )
    partials = kern(features, labels, centers)
    return jnp.sum(partials) / BATCH


def kernel(features, labels, centers):
    return _center_loss(features, labels, centers)
